# baseline (speedup 1.0000x reference)
import functools

import jax
import jax.numpy as jnp
import numpy as np
from jax import lax
from jax.experimental import pallas as pl
from jax.experimental.pallas import tpu as pltpu
from jax.experimental.pallas import tpu_sc as plsc

K = 16
D = 64
NC = 2
NS = 16
NW = NC * NS
LANES = 16

_F32_MAX = np.float32(3.4e38)
_I32_MAX = np.int32(2147483647)


def _rsqrt(x):
    i = lax.bitcast_convert_type(x, jnp.int32)
    y = lax.bitcast_convert_type(np.int32(0x5F3759DF) - (i >> 1), jnp.float32)
    half = np.float32(0.5)
    threehalf = np.float32(1.5)
    for _ in range(3):
        y = y * (threehalf - half * x * y * y)
    return y


def _make_sc_stage(n_rows):
    rows_per_w = n_rows // NW
    assert rows_per_w * NW == n_rows
    C = 976
    n_full = rows_per_w // C
    tail = rows_per_w - n_full * C
    assert n_full % 2 == 0
    groups_full = C // LANES

    mesh = plsc.VectorSubcoreMesh(core_axis_name="c", subcore_axis_name="s")

    @functools.partial(
        pl.kernel,
        mesh=mesh,
        compiler_params=pltpu.CompilerParams(needs_layout_passes=False),
        out_type=(
            jax.ShapeDtypeStruct((NW, K), jnp.float32),
            jax.ShapeDtypeStruct((NW, K), jnp.int32),
        ),
        scratch_types=[
            pltpu.VMEM((C * D,), jnp.float32),
            pltpu.VMEM((C * D,), jnp.float32),
            pltpu.VMEM((D,), jnp.float32),
            pltpu.VMEM((K,), jnp.float32),
            pltpu.VMEM((K,), jnp.int32),
            pltpu.SemaphoreType.DMA,
            pltpu.SemaphoreType.DMA,
        ],
    )
    def sc_stage(q_hbm, db_hbm, outv_hbm, outi_hbm,
                 buf0, buf1, qbuf, stgv, stgi, sem0, sem1):
        wid = lax.axis_index("s") * NC + lax.axis_index("c")
        base = wid * rows_per_w
        iota = lax.iota(jnp.int32, LANES)

        pltpu.sync_copy(q_hbm, qbuf)

        qvs = [qbuf[pl.ds(c * LANES, LANES)] for c in range(D // LANES)]
        qsq = jnp.zeros((LANES,), jnp.float32)
        for qv in qvs:
            qsq = qsq + qv * qv
        for sh in (8, 4, 2, 1):
            qsq = qsq + qsq.at[iota ^ sh].get(mode="promise_in_bounds")
        q2v = qsq
        qnv = q2v * _rsqrt(q2v)

        def merge(bv, bi, tv, d, gi):
            cnt = plsc.all_reduce_population_count(d < tv)
            pred = cnt[0] > 0

            def do(args):
                bv, bi, d, gi = args
                ds_, is_ = plsc.sort_key_val(d, gi, descending=True)
                keep = bv <= ds_
                mv = jnp.where(keep, bv, ds_)
                mi = jnp.where(keep, bi, is_)
                sv, si = plsc.sort_key_val(mv, mi)
                stv = jnp.zeros((LANES,), jnp.float32) + sv[K - 1]
                return sv, si, stv

            def skip(args):
                bv, bi, _, _ = args
                return bv, bi, tv

            return lax.cond(pred, do, skip, (bv, bi, d, gi))

        def group_dist(buf, rows):
            rbase = rows * D
            dot = jnp.zeros((LANES,), jnp.float32)
            nrm = jnp.zeros((LANES,), jnp.float32)
            for k in range(D):
                col = plsc.load_gather(buf, [rbase + k])
                qk = qvs[k // LANES][k % LANES]
                dot = dot + qk * col
                nrm = nrm + col * col
            dn = nrm * _rsqrt(nrm)
            denom = jnp.maximum(qnv * dn, np.float32(1e-8))
            return np.float32(1.0) - dot / denom

        def process_chunk(buf, chunk_base, bv, bi, tv):
            def g_body(g, carry):
                bv, bi, tv = carry
                rows = g * LANES + iota
                d = group_dist(buf, rows)
                gi = chunk_base + rows
                return merge(bv, bi, tv, d, gi)
            return lax.fori_loop(0, groups_full, g_body, (bv, bi, tv))

        def start(c, buf, sem):
            return pltpu.async_copy(
                db_hbm.at[pl.ds((base + c * C) * D, C * D)], buf, sem)

        bv0 = jnp.full((K,), _F32_MAX, jnp.float32)
        bi0 = jnp.zeros((K,), jnp.int32)
        tv0 = jnp.full((LANES,), _F32_MAX, jnp.float32)

        start(0, buf0, sem0)
        start(1, buf1, sem1)

        def pair_body(i, carry):
            bv, bi, tv = carry
            c0 = 2 * i
            pltpu.make_async_copy(db_hbm.at[pl.ds(0, C * D)], buf0, sem0).wait()
            bv, bi, tv = process_chunk(buf0, base + c0 * C, bv, bi, tv)

            @pl.when(i < n_full // 2 - 1)
            def _():
                start(c0 + 2, buf0, sem0)

            pltpu.make_async_copy(db_hbm.at[pl.ds(0, C * D)], buf1, sem1).wait()
            bv, bi, tv = process_chunk(buf1, base + c0 * C + C, bv, bi, tv)

            @pl.when(i < n_full // 2 - 1)
            def _():
                start(c0 + 3, buf1, sem1)

            return bv, bi, tv

        bv, bi, tv = lax.fori_loop(0, n_full // 2, pair_body, (bv0, bi0, tv0))

        if tail > 0:
            t0 = base + n_full * C
            pltpu.sync_copy(db_hbm.at[pl.ds(t0 * D, tail * D)],
                            buf0.at[pl.ds(0, tail * D)])
            n_tail_groups = (tail + LANES - 1) // LANES
            for g in range(n_tail_groups):
                rows = g * LANES + iota
                d = group_dist(buf0, rows)
                valid = rows < tail
                d = jnp.where(valid, d, _F32_MAX)
                gi = t0 + rows
                bv, bi, tv = merge(bv, bi, tv, d, gi)

        stgv[...] = bv
        stgi[...] = bi
        pltpu.sync_copy(stgv, outv_hbm.at[wid])
        pltpu.sync_copy(stgi, outi_hbm.at[wid])

    return sc_stage


def _final_merge(v_ref, i_ref, d_ref, x_ref):
    v = v_ref[...]
    ids = i_ref[...]
    outd = jnp.zeros((1, K), jnp.float32)
    outi = jnp.zeros((1, K), jnp.int32)
    col = lax.broadcasted_iota(jnp.int32, (1, K), 1)
    for j in range(K):
        m = jnp.min(v)
        sel = v == m
        w = jnp.min(jnp.where(sel, ids, _I32_MAX))
        outd = jnp.where(col == j, m, outd)
        outi = jnp.where(col == j, w, outi)
        v = jnp.where(sel & (ids == w), _F32_MAX, v)
    d_ref[...] = outd
    x_ref[...] = outi


@jax.jit
def kernel(query, database):
    n = database.shape[0]
    db_flat = database.reshape(-1)
    q_flat = query.reshape(-1)
    cand_v, cand_i = _make_sc_stage(n)(q_flat, db_flat)
    d, idx = pl.pallas_call(
        _final_merge,
        out_shape=(
            jax.ShapeDtypeStruct((1, K), jnp.float32),
            jax.ShapeDtypeStruct((1, K), jnp.int32),
        ),
    )(cand_v, cand_i)
    return d, idx


# speedup vs baseline: 2.0509x; 2.0509x over previous
import functools

import jax
import jax.numpy as jnp
import numpy as np
from jax import lax
from jax.experimental import pallas as pl
from jax.experimental.pallas import tpu as pltpu
from jax.experimental.pallas import tpu_sc as plsc

K = 16
D = 64
NC = 2
NS = 16
NW = NC * NS
LANES = 16

_F32_MAX = np.float32(3.4e38)
_I32_MAX = np.int32(2147483647)


def _rsqrt(x):
    i = lax.bitcast_convert_type(x, jnp.int32)
    y = lax.bitcast_convert_type(np.int32(0x5F3759DF) - (i >> 1), jnp.float32)
    half = np.float32(0.5)
    threehalf = np.float32(1.5)
    for _ in range(3):
        y = y * (threehalf - half * x * y * y)
    return y


def _make_sc_stage(n_rows):
    rows_per_w = n_rows // NW
    assert rows_per_w * NW == n_rows
    C = 976
    n_full = rows_per_w // C
    tail = rows_per_w - n_full * C
    assert n_full % 2 == 0
    groups_full = C // LANES

    mesh = plsc.VectorSubcoreMesh(core_axis_name="c", subcore_axis_name="s")

    @functools.partial(
        pl.kernel,
        mesh=mesh,
        compiler_params=pltpu.CompilerParams(
            needs_layout_passes=False, use_tc_tiling_on_sc=False),
        out_type=(
            jax.ShapeDtypeStruct((NW, K), jnp.float32),
            jax.ShapeDtypeStruct((NW, K), jnp.int32),
        ),
        scratch_types=[
            pltpu.VMEM((C * D,), jnp.float32),
            pltpu.VMEM((C * D,), jnp.float32),
            pltpu.VMEM((D,), jnp.float32),
            pltpu.VMEM((K,), jnp.float32),
            pltpu.VMEM((K,), jnp.int32),
            pltpu.SemaphoreType.DMA,
            pltpu.SemaphoreType.DMA,
        ],
    )
    def sc_stage(q_hbm, db_hbm, outv_hbm, outi_hbm,
                 buf0, buf1, qbuf, stgv, stgi, sem0, sem1):
        wid = lax.axis_index("s") * NC + lax.axis_index("c")
        base = wid * rows_per_w
        iota = lax.iota(jnp.int32, LANES)

        pltpu.sync_copy(q_hbm, qbuf)

        qvs = [qbuf[pl.ds(c * LANES, LANES)] for c in range(D // LANES)]
        qsq = jnp.zeros((LANES,), jnp.float32)
        for qv in qvs:
            qsq = qsq + qv * qv
        for sh in (8, 4, 2, 1):
            qsq = qsq + qsq.at[iota ^ sh].get(mode="promise_in_bounds")
        q2v = qsq
        qnv = q2v * _rsqrt(q2v)

        def merge(bv, bi, tv, d, gi):
            cnt = plsc.all_reduce_population_count(d < tv)
            pred = cnt[0] > 0

            def do(args):
                bv, bi, d, gi = args
                ds_, is_ = plsc.sort_key_val(d, gi, descending=True)
                keep = bv <= ds_
                mv = jnp.where(keep, bv, ds_)
                mi = jnp.where(keep, bi, is_)
                sv, si = plsc.sort_key_val(mv, mi)
                stv = jnp.zeros((LANES,), jnp.float32) + sv[K - 1]
                return sv, si, stv

            def skip(args):
                bv, bi, _, _ = args
                return bv, bi, tv

            return lax.cond(pred, do, skip, (bv, bi, d, gi))

        def group_dist(buf, g):
            dot = jnp.zeros((LANES,), jnp.float32)
            nrm = jnp.zeros((LANES,), jnp.float32)
            for r in range(LANES):
                rb = (g * LANES + r) * D
                xs = [buf[pl.ds(rb + c * LANES, LANES)]
                      for c in range(D // LANES)]
                pd = qvs[0] * xs[0]
                pn = xs[0] * xs[0]
                for c in range(1, D // LANES):
                    pd = pd + qvs[c] * xs[c]
                    pn = pn + xs[c] * xs[c]
                m = iota == r
                dot = jnp.where(m, jnp.sum(pd), dot)
                nrm = jnp.where(m, jnp.sum(pn), nrm)
            dn = nrm * _rsqrt(nrm)
            denom = jnp.maximum(qnv * dn, np.float32(1e-8))
            return np.float32(1.0) - dot / denom

        def process_chunk(buf, chunk_base, bv, bi, tv):
            def g_body(g, carry):
                bv, bi, tv = carry
                d = group_dist(buf, g)
                gi = chunk_base + g * LANES + iota
                return merge(bv, bi, tv, d, gi)
            return lax.fori_loop(0, groups_full, g_body, (bv, bi, tv))

        def start(c, buf, sem):
            return pltpu.async_copy(
                db_hbm.at[pl.ds((base + c * C) * D, C * D)], buf, sem)

        bv0 = jnp.full((K,), _F32_MAX, jnp.float32)
        bi0 = jnp.zeros((K,), jnp.int32)
        tv0 = jnp.full((LANES,), _F32_MAX, jnp.float32)

        start(0, buf0, sem0)
        start(1, buf1, sem1)

        def pair_body(i, carry):
            bv, bi, tv = carry
            c0 = 2 * i
            pltpu.make_async_copy(db_hbm.at[pl.ds(0, C * D)], buf0,
                                  sem0).wait()
            bv, bi, tv = process_chunk(buf0, base + c0 * C, bv, bi, tv)

            @pl.when(i < n_full // 2 - 1)
            def _():
                start(c0 + 2, buf0, sem0)

            pltpu.make_async_copy(db_hbm.at[pl.ds(0, C * D)], buf1,
                                  sem1).wait()
            bv, bi, tv = process_chunk(buf1, base + c0 * C + C, bv, bi, tv)

            @pl.when(i < n_full // 2 - 1)
            def _():
                start(c0 + 3, buf1, sem1)

            return bv, bi, tv

        bv, bi, tv = lax.fori_loop(0, n_full // 2, pair_body, (bv0, bi0, tv0))

        if tail > 0:
            t0 = base + n_full * C
            pltpu.sync_copy(db_hbm.at[pl.ds(t0 * D, tail * D)],
                            buf0.at[pl.ds(0, tail * D)])
            n_tail_groups = (tail + LANES - 1) // LANES
            for g in range(n_tail_groups):
                d = group_dist(buf0, g)
                lrows = g * LANES + iota
                valid = lrows < tail
                d = jnp.where(valid, d, _F32_MAX)
                gi = t0 + lrows
                bv, bi, tv = merge(bv, bi, tv, d, gi)

        stgv[...] = bv
        stgi[...] = bi
        pltpu.sync_copy(stgv, outv_hbm.at[wid])
        pltpu.sync_copy(stgi, outi_hbm.at[wid])

    return sc_stage


def _final_merge(v_ref, i_ref, d_ref, x_ref):
    v = v_ref[...]
    ids = i_ref[...]
    outd = jnp.zeros((1, K), jnp.float32)
    outi = jnp.zeros((1, K), jnp.int32)
    col = lax.broadcasted_iota(jnp.int32, (1, K), 1)
    for j in range(K):
        m = jnp.min(v)
        sel = v == m
        w = jnp.min(jnp.where(sel, ids, _I32_MAX))
        outd = jnp.where(col == j, m, outd)
        outi = jnp.where(col == j, w, outi)
        v = jnp.where(sel & (ids == w), _F32_MAX, v)
    d_ref[...] = outd
    x_ref[...] = outi


@jax.jit
def kernel(query, database):
    n = database.shape[0]
    q_flat = query.reshape(-1)
    db_flat = database.reshape(-1)
    cand_v, cand_i = _make_sc_stage(n)(q_flat, db_flat)
    d, idx = pl.pallas_call(
        _final_merge,
        out_shape=(
            jax.ShapeDtypeStruct((1, K), jnp.float32),
            jax.ShapeDtypeStruct((1, K), jnp.int32),
        ),
    )(cand_v, cand_i)
    return d, idx
